# baseline (speedup 1.0000x reference)
import functools

import jax
import jax.numpy as jnp
from jax import lax
from jax.experimental import pallas as pl
from jax.experimental.pallas import tpu as pltpu
from jax.experimental.pallas import tpu_sc as plsc

NUM_LAYERS = 4
HIDDEN = 256
N_NODES = 10000
N_EDGES = 160000

NB = 1000
EB = 1600
NGRID = N_NODES // NB
EGRID = N_EDGES // EB

NCORES = 2
NSUB = 16
NW = NCORES * NSUB

GC = 64
G_CHUNKS = N_EDGES // GC
G_ITERS = -(-G_CHUNKS // NW)

SC_C = 64
S_CHUNKS = N_EDGES // SC_C
S_ITERS = -(-S_CHUNKS // NSUB)
NSLOT = 6
CPASS = 128
RPT = 624
RPT_LAST = N_NODES - (NSUB - 1) * RPT


def _proj_body(h_ref, w_ref, b_ref, a1_ref, tsrc_ref, tdst_ref):
    r = (
        jnp.dot(h_ref[...], w_ref[...], preferred_element_type=jnp.float32)
        + b_ref[...]
    )
    a1_ref[...] = r[:, 0:HIDDEN]
    tsrc_ref[...] = r[:, HIDDEN:3 * HIDDEN].astype(jnp.bfloat16)
    tdst_ref[...] = r[:, 3 * HIDDEN:5 * HIDDEN].astype(jnp.bfloat16)


def _node_proj(h, wcat, bcat):
    return pl.pallas_call(
        _proj_body,
        grid=(NGRID,),
        in_specs=[
            pl.BlockSpec((NB, HIDDEN), lambda i: (i, 0)),
            pl.BlockSpec((HIDDEN, 5 * HIDDEN), lambda i: (0, 0)),
            pl.BlockSpec((1, 5 * HIDDEN), lambda i: (0, 0)),
        ],
        out_specs=[
            pl.BlockSpec((NB, HIDDEN), lambda i: (i, 0)),
            pl.BlockSpec((NB, 2 * HIDDEN), lambda i: (i, 0)),
            pl.BlockSpec((NB, 2 * HIDDEN), lambda i: (i, 0)),
        ],
        out_shape=[
            jax.ShapeDtypeStruct((N_NODES, HIDDEN), jnp.float32),
            jax.ShapeDtypeStruct((N_NODES, 2 * HIDDEN), jnp.bfloat16),
            jax.ShapeDtypeStruct((N_NODES, 2 * HIDDEN), jnp.bfloat16),
        ],
    )(h, wcat, bcat)


def _edge_a_body(e_ref, g1_ref, g2_ref, w_ref, b_ref, t_ref, stats_ref):
    t = (
        jnp.dot(e_ref[...], w_ref[...], preferred_element_type=jnp.float32)
        + b_ref[...]
        + g1_ref[...].astype(jnp.float32)
        + g2_ref[...].astype(jnp.float32)
    )
    t_ref[...] = t.astype(jnp.bfloat16)
    part = jnp.concatenate(
        [
            jnp.sum(t, axis=0, keepdims=True),
            jnp.sum(t * t, axis=0, keepdims=True),
            jnp.zeros((6, HIDDEN), jnp.float32),
        ],
        axis=0,
    )

    @pl.when(pl.program_id(0) == 0)
    def _init():
        stats_ref[...] = part

    @pl.when(pl.program_id(0) != 0)
    def _acc():
        stats_ref[...] += part


def _edge_a(e, gs, gd, w5t, b5):
    return pl.pallas_call(
        _edge_a_body,
        grid=(EGRID,),
        in_specs=[
            pl.BlockSpec((EB, HIDDEN), lambda i: (i, 0)),
            pl.BlockSpec((EB, HIDDEN), lambda i: (i, 1)),
            pl.BlockSpec((EB, HIDDEN), lambda i: (i, 1)),
            pl.BlockSpec((HIDDEN, HIDDEN), lambda i: (0, 0)),
            pl.BlockSpec((1, HIDDEN), lambda i: (0, 0)),
        ],
        out_specs=[
            pl.BlockSpec((EB, HIDDEN), lambda i: (i, 0)),
            pl.BlockSpec((8, HIDDEN), lambda i: (0, 0)),
        ],
        out_shape=[
            jax.ShapeDtypeStruct((N_EDGES, HIDDEN), jnp.bfloat16),
            jax.ShapeDtypeStruct((8, HIDDEN), jnp.float32),
        ],
    )(e, gs, gd, w5t, b5)


def _bn_apply(t, stats_ref, count, gamma_ref, beta_ref):
    mu = stats_ref[0:1, :] / count
    msq = stats_ref[1:2, :] / count
    var = msq - mu * mu
    inv = jax.lax.rsqrt(var + 1e-5)
    return gamma_ref[...] * (t - mu) * inv + beta_ref[...]


def _edge_b_body(t_ref, e_ref, g3_ref, g4_ref, stats_ref, gamma_ref, beta_ref,
                 en_ref, sig_ref, mf_ref, mb_ref):
    x = _bn_apply(t_ref[...].astype(jnp.float32), stats_ref, float(N_EDGES),
                  gamma_ref, beta_ref)
    en = e_ref[...] + jnp.maximum(x, 0.0)
    en_ref[...] = en
    sig = jax.nn.sigmoid(en)
    sig_ref[...] = sig
    mf_ref[...] = g3_ref[...].astype(jnp.float32) * sig
    mb_ref[...] = g4_ref[...].astype(jnp.float32) * sig


def _edge_b(t, e, gs, gd, stats, gamma, beta):
    return pl.pallas_call(
        _edge_b_body,
        grid=(EGRID,),
        in_specs=[
            pl.BlockSpec((EB, HIDDEN), lambda i: (i, 0)),
            pl.BlockSpec((EB, HIDDEN), lambda i: (i, 0)),
            pl.BlockSpec((EB, HIDDEN), lambda i: (i, 0)),
            pl.BlockSpec((EB, HIDDEN), lambda i: (i, 0)),
            pl.BlockSpec((8, HIDDEN), lambda i: (0, 0)),
            pl.BlockSpec((1, HIDDEN), lambda i: (0, 0)),
            pl.BlockSpec((1, HIDDEN), lambda i: (0, 0)),
        ],
        out_specs=[pl.BlockSpec((EB, HIDDEN), lambda i: (i, 0))] * 4,
        out_shape=[jax.ShapeDtypeStruct((N_EDGES, HIDDEN), jnp.float32)] * 4,
    )(t, e, gs, gd, stats, gamma, beta)


def _node_a_body(a1_ref, shf_ref, sf_ref, shb_ref, sb_ref, ht_ref, stats_ref):
    ht = (
        a1_ref[...]
        + shf_ref[...] / (sf_ref[...] + 1e-6)
        + shb_ref[...] / (sb_ref[...] + 1e-6)
    )
    ht_ref[...] = ht
    part = jnp.concatenate(
        [
            jnp.sum(ht, axis=0, keepdims=True),
            jnp.sum(ht * ht, axis=0, keepdims=True),
            jnp.zeros((6, HIDDEN), jnp.float32),
        ],
        axis=0,
    )

    @pl.when(pl.program_id(0) == 0)
    def _init():
        stats_ref[...] = part

    @pl.when(pl.program_id(0) != 0)
    def _acc():
        stats_ref[...] += part


def _node_a(a1h, shf, sf, shb, sb):
    return pl.pallas_call(
        _node_a_body,
        grid=(NGRID,),
        in_specs=[pl.BlockSpec((NB, HIDDEN), lambda i: (i, 0))] * 5,
        out_specs=[
            pl.BlockSpec((NB, HIDDEN), lambda i: (i, 0)),
            pl.BlockSpec((8, HIDDEN), lambda i: (0, 0)),
        ],
        out_shape=[
            jax.ShapeDtypeStruct((N_NODES, HIDDEN), jnp.float32),
            jax.ShapeDtypeStruct((8, HIDDEN), jnp.float32),
        ],
    )(a1h, shf, sf, shb, sb)


def _node_b_body(ht_ref, h_ref, stats_ref, gamma_ref, beta_ref, out_ref):
    x = _bn_apply(ht_ref[...], stats_ref, float(N_NODES), gamma_ref, beta_ref)
    out_ref[...] = h_ref[...] + jnp.maximum(x, 0.0)


def _node_b(ht, h, stats, gamma, beta):
    return pl.pallas_call(
        _node_b_body,
        grid=(NGRID,),
        in_specs=[
            pl.BlockSpec((NB, HIDDEN), lambda i: (i, 0)),
            pl.BlockSpec((NB, HIDDEN), lambda i: (i, 0)),
            pl.BlockSpec((8, HIDDEN), lambda i: (0, 0)),
            pl.BlockSpec((1, HIDDEN), lambda i: (0, 0)),
            pl.BlockSpec((1, HIDDEN), lambda i: (0, 0)),
        ],
        out_specs=pl.BlockSpec((NB, HIDDEN), lambda i: (i, 0)),
        out_shape=jax.ShapeDtypeStruct((N_NODES, HIDDEN), jnp.float32),
    )(ht, h, stats, gamma, beta)


_SC_MESH = plsc.VectorSubcoreMesh(core_axis_name="c", subcore_axis_name="s")


@functools.partial(
    pl.kernel,
    out_type=[
        jax.ShapeDtypeStruct((N_EDGES, HIDDEN), jnp.int32),
        jax.ShapeDtypeStruct((N_EDGES, HIDDEN), jnp.int32),
    ],
    mesh=_SC_MESH,
    scratch_types=[
        pltpu.VMEM((GC,), jnp.int32),
        pltpu.VMEM((GC,), jnp.int32),
        pltpu.VMEM((GC, HIDDEN), jnp.int32),
        pltpu.VMEM((GC, HIDDEN), jnp.int32),
        pltpu.SemaphoreType.DMA,
        pltpu.SemaphoreType.DMA,
    ],
)
def _sc_gather(tsrc_hbm, tdst_hbm, src_hbm, dst_hbm, gs_hbm, gd_hbm,
               idx1_v, idx2_v, rows1_v, rows2_v, sem1, sem2):
    wid = lax.axis_index("s") * NCORES + lax.axis_index("c")

    def chunk(k, _):
        cid = wid + k * NW

        @pl.when(cid < G_CHUNKS)
        def _do():
            eoff = pl.multiple_of(cid * GC, GC)
            pltpu.sync_copy(src_hbm.at[pl.ds(eoff, GC)], idx1_v)
            pltpu.sync_copy(dst_hbm.at[pl.ds(eoff, GC)], idx2_v)
            c1 = pltpu.async_copy(tsrc_hbm.at[idx1_v], rows1_v, sem1)
            c2 = pltpu.async_copy(tdst_hbm.at[idx2_v], rows2_v, sem2)
            c1.wait()
            pltpu.sync_copy(rows1_v, gs_hbm.at[pl.ds(eoff, GC)])
            c2.wait()
            pltpu.sync_copy(rows2_v, gd_hbm.at[pl.ds(eoff, GC)])

        return 0

    lax.fori_loop(0, G_ITERS, chunk, 0)


@functools.partial(
    pl.kernel,
    out_type=[jax.ShapeDtypeStruct((N_NODES, HIDDEN), jnp.float32)] * 4,
    mesh=_SC_MESH,
    scratch_types=[
        pltpu.VMEM((8, CPASS), jnp.float32),
        [pltpu.VMEM((SC_C,), jnp.int32)] * NSLOT,
        [pltpu.VMEM((SC_C, CPASS), jnp.float32)] * NSLOT,
        [pltpu.SemaphoreType.DMA] * NSLOT,
        [pltpu.SemaphoreType.DMA] * NSLOT,
        pltpu.VMEM_SHARED((N_NODES, CPASS), jnp.float32),
    ],
)
def _sc_scatter(msgf_hbm, msgb_hbm, sig_hbm, dst_hbm, src_hbm,
                shf_hbm, shb_hbm, sf_hbm, sb_hbm,
                zbuf_v, idx_s, buf_s, ldsem_s, adsem_s, acc_sh):
    cid = lax.axis_index("c")
    sid = lax.axis_index("s")
    r0 = pl.multiple_of(sid * RPT, 8)
    last = sid == NSUB - 1

    def zrow(i, _):
        for j in range(CPASS // 16):
            zbuf_v[i, pl.ds(j * 16, 16)] = jnp.zeros((16,), jnp.float32)
        return 0

    lax.fori_loop(0, 8, zrow, 0)
    nz = jnp.where(last, RPT_LAST // 8, RPT // 8)

    def _valid(k):
        return jnp.logical_and(k >= 0, sid + k * NSUB < S_CHUNKS)

    def _eoff(k):
        return pl.multiple_of((sid + k * NSUB) * SC_C, SC_C)

    for p in range(4):
        msg_hbm = msgf_hbm if p < 2 else msgb_hbm
        idx_hbm = dst_hbm if p < 2 else src_hbm
        outs = (shf_hbm, sf_hbm) if p < 2 else (shb_hbm, sb_hbm)
        c0 = (p % 2) * CPASS

        def zcopy(i, _):
            pltpu.sync_copy(zbuf_v, acc_sh.at[pl.ds(r0 + i * 8, 8)])
            return 0

        lax.fori_loop(0, nz, zcopy, 0)
        plsc.subcore_barrier()

        def _start_load(k, s, msg_hbm=msg_hbm, idx_hbm=idx_hbm, c0=c0):
            eoff = _eoff(k)
            pltpu.async_copy(idx_hbm.at[pl.ds(eoff, SC_C)], idx_s[s],
                             ldsem_s[s])

            @pl.when(cid == 0)
            def _ld_msg():
                pltpu.async_copy(
                    msg_hbm.at[pl.ds(eoff, SC_C), pl.ds(c0, CPASS)],
                    buf_s[s], ldsem_s[s])

            @pl.when(cid == 1)
            def _ld_sig():
                pltpu.async_copy(
                    sig_hbm.at[pl.ds(eoff, SC_C), pl.ds(c0, CPASS)],
                    buf_s[s], ldsem_s[s])

        def _wait_load(k, s, msg_hbm=msg_hbm, idx_hbm=idx_hbm, c0=c0):
            eoff = _eoff(k)
            pltpu.make_async_copy(idx_hbm.at[pl.ds(eoff, SC_C)], idx_s[s],
                                  ldsem_s[s]).wait()
            pltpu.make_async_copy(
                msg_hbm.at[pl.ds(eoff, SC_C), pl.ds(c0, CPASS)],
                buf_s[s], ldsem_s[s]).wait()

        def _wait_add(s):
            pltpu.make_async_copy(buf_s[s], acc_sh.at[idx_s[s]],
                                  adsem_s[s]).wait()

        def step(g, _, _start_load=_start_load, _wait_load=_wait_load):
            for s in range(NSLOT):
                k = g * NSLOT + s

                @pl.when(_valid(k - NSLOT))
                def _drain(s=s):
                    _wait_add(s)

                @pl.when(_valid(k))
                def _load(k=k, s=s):
                    _start_load(k, s)

                j = k - (NSLOT - 1)
                sj = (s + 1) % NSLOT

                @pl.when(_valid(j))
                def _process(j=j, sj=sj):
                    _wait_load(j, sj)
                    pltpu.async_copy(buf_s[sj], acc_sh.at[idx_s[sj]],
                                     adsem_s[sj], add=True)

            return 0

        n_groups = -(-(S_ITERS + NSLOT - 1) // NSLOT)
        lax.fori_loop(0, n_groups, step, 0)
        k_tail = NSLOT * (n_groups - 1)

        @pl.when(_valid(k_tail))
        def _drain_tail():
            _wait_add(k_tail % NSLOT)

        plsc.subcore_barrier()

        for c in range(NCORES):
            @pl.when(jnp.logical_and(cid == c, jnp.logical_not(last)))
            def _wb0(c=c, outs=outs, c0=c0):
                pltpu.sync_copy(
                    acc_sh.at[pl.ds(r0, RPT)],
                    outs[c].at[pl.ds(r0, RPT), pl.ds(c0, CPASS)])

            @pl.when(jnp.logical_and(cid == c, last))
            def _wb1(c=c, outs=outs, c0=c0):
                pltpu.sync_copy(
                    acc_sh.at[pl.ds(r0, RPT_LAST)],
                    outs[c].at[pl.ds(r0, RPT_LAST), pl.ds(c0, CPASS)])

        plsc.subcore_barrier()


def kernel(h, e, edge_index, W, b, bn_gamma, bn_beta):
    src, dst = edge_index[0], edge_index[1]

    for l in range(NUM_LAYERS):
        order = (0, 1, 3, 2, 4)
        wcat = jnp.concatenate([W[l, i].T for i in order], axis=1)
        bcat = jnp.concatenate([b[l, i] for i in order]).reshape(1, 5 * HIDDEN)
        a1h, tsrc, tdst = _node_proj(h, wcat, bcat)

        tsrc_i = jax.lax.bitcast_convert_type(
            tsrc.reshape(N_NODES, HIDDEN, 2), jnp.int32)
        tdst_i = jax.lax.bitcast_convert_type(
            tdst.reshape(N_NODES, HIDDEN, 2), jnp.int32)
        gs_i, gd_i = _sc_gather(tsrc_i, tdst_i, src, dst)
        gs = jax.lax.bitcast_convert_type(gs_i, jnp.bfloat16).reshape(
            N_EDGES, 2 * HIDDEN)
        gd = jax.lax.bitcast_convert_type(gd_i, jnp.bfloat16).reshape(
            N_EDGES, 2 * HIDDEN)

        t, estats = _edge_a(e, gs, gd, W[l, 5].T, b[l, 5].reshape(1, HIDDEN))
        e, sigma, msgf, msgb = _edge_b(
            t, e, gs, gd, estats,
            bn_gamma[l, 1].reshape(1, HIDDEN), bn_beta[l, 1].reshape(1, HIDDEN),
        )

        shf, shb, sf, sb = _sc_scatter(msgf, msgb, sigma, dst, src)

        ht, nstats = _node_a(a1h, shf, sf, shb, sb)
        h = _node_b(
            ht, h, nstats,
            bn_gamma[l, 0].reshape(1, HIDDEN), bn_beta[l, 0].reshape(1, HIDDEN),
        )

    return h, e


# speedup vs baseline: 3.1301x; 3.1301x over previous
import functools

import jax
import jax.numpy as jnp
from jax import lax
from jax.experimental import pallas as pl
from jax.experimental.pallas import tpu as pltpu
from jax.experimental.pallas import tpu_sc as plsc

NUM_LAYERS = 4
HIDDEN = 256
N_NODES = 10000
N_EDGES = 160000

NB = 1000
EB = 1600
NGRID = N_NODES // NB
EGRID = N_EDGES // EB

NCORES = 2
NSUB = 16
NW = NCORES * NSUB

GC = 64
G_CHUNKS = N_EDGES // GC
G_ITERS = -(-G_CHUNKS // NW)

SC_C = 64
S_CHUNKS = N_EDGES // SC_C
S_ITERS = -(-S_CHUNKS // NSUB)
NSLOT = 6
CPASS = 128
RPT = 624
RPT_LAST = N_NODES - (NSUB - 1) * RPT


def _pack_bf16_pair(lo_f32, hi_f32):
    lo = jax.lax.bitcast_convert_type(
        lo_f32.astype(jnp.bfloat16), jnp.uint16).astype(jnp.uint32)
    hi = jax.lax.bitcast_convert_type(
        hi_f32.astype(jnp.bfloat16), jnp.uint16).astype(jnp.uint32)
    return jax.lax.bitcast_convert_type((hi << 16) | lo, jnp.int32)


def _unpack_lo(w_i32):
    u = jax.lax.bitcast_convert_type(w_i32, jnp.uint32)
    return jax.lax.bitcast_convert_type(
        (u & 0xFFFF).astype(jnp.uint16), jnp.bfloat16).astype(jnp.float32)


def _unpack_hi(w_i32):
    u = jax.lax.bitcast_convert_type(w_i32, jnp.uint32)
    return jax.lax.bitcast_convert_type(
        (u >> 16).astype(jnp.uint16), jnp.bfloat16).astype(jnp.float32)


def _proj_body(h_ref, w_ref, b_ref, a1_ref, tsrc_ref, tdst_ref):
    r = (
        jnp.dot(h_ref[...], w_ref[...], preferred_element_type=jnp.float32)
        + b_ref[...]
    )
    a1_ref[...] = r[:, 0:HIDDEN]
    tsrc_ref[...] = _pack_bf16_pair(r[:, HIDDEN:2 * HIDDEN],
                                    r[:, 2 * HIDDEN:3 * HIDDEN])
    tdst_ref[...] = _pack_bf16_pair(r[:, 3 * HIDDEN:4 * HIDDEN],
                                    r[:, 4 * HIDDEN:5 * HIDDEN])


def _node_proj(h, wcat, bcat):
    return pl.pallas_call(
        _proj_body,
        grid=(NGRID,),
        in_specs=[
            pl.BlockSpec((NB, HIDDEN), lambda i: (i, 0)),
            pl.BlockSpec((HIDDEN, 5 * HIDDEN), lambda i: (0, 0)),
            pl.BlockSpec((1, 5 * HIDDEN), lambda i: (0, 0)),
        ],
        out_specs=[
            pl.BlockSpec((NB, HIDDEN), lambda i: (i, 0)),
            pl.BlockSpec((NB, HIDDEN), lambda i: (i, 0)),
            pl.BlockSpec((NB, HIDDEN), lambda i: (i, 0)),
        ],
        out_shape=[
            jax.ShapeDtypeStruct((N_NODES, HIDDEN), jnp.float32),
            jax.ShapeDtypeStruct((N_NODES, HIDDEN), jnp.int32),
            jax.ShapeDtypeStruct((N_NODES, HIDDEN), jnp.int32),
        ],
    )(h, wcat, bcat)


def _edge_a_body(e_ref, g1_ref, g2_ref, w_ref, b_ref, t_ref, stats_ref):
    t = (
        jnp.dot(e_ref[...], w_ref[...], preferred_element_type=jnp.float32)
        + b_ref[...]
        + _unpack_hi(g1_ref[...])
        + _unpack_hi(g2_ref[...])
    )
    t_ref[...] = t.astype(jnp.bfloat16)
    part = jnp.concatenate(
        [
            jnp.sum(t, axis=0, keepdims=True),
            jnp.sum(t * t, axis=0, keepdims=True),
            jnp.zeros((6, HIDDEN), jnp.float32),
        ],
        axis=0,
    )

    @pl.when(pl.program_id(0) == 0)
    def _init():
        stats_ref[...] = part

    @pl.when(pl.program_id(0) != 0)
    def _acc():
        stats_ref[...] += part


def _edge_a(e, gs, gd, w5t, b5):
    return pl.pallas_call(
        _edge_a_body,
        grid=(EGRID,),
        in_specs=[
            pl.BlockSpec((EB, HIDDEN), lambda i: (i, 0)),
            pl.BlockSpec((EB, HIDDEN), lambda i: (i, 0)),
            pl.BlockSpec((EB, HIDDEN), lambda i: (i, 0)),
            pl.BlockSpec((HIDDEN, HIDDEN), lambda i: (0, 0)),
            pl.BlockSpec((1, HIDDEN), lambda i: (0, 0)),
        ],
        out_specs=[
            pl.BlockSpec((EB, HIDDEN), lambda i: (i, 0)),
            pl.BlockSpec((8, HIDDEN), lambda i: (0, 0)),
        ],
        out_shape=[
            jax.ShapeDtypeStruct((N_EDGES, HIDDEN), jnp.bfloat16),
            jax.ShapeDtypeStruct((8, HIDDEN), jnp.float32),
        ],
    )(e, gs, gd, w5t, b5)


def _bn_apply(t, stats_ref, count, gamma_ref, beta_ref):
    mu = stats_ref[0:1, :] / count
    msq = stats_ref[1:2, :] / count
    var = msq - mu * mu
    inv = jax.lax.rsqrt(var + 1e-5)
    return gamma_ref[...] * (t - mu) * inv + beta_ref[...]


def _edge_b_body(t_ref, e_ref, g3_ref, g4_ref, stats_ref, gamma_ref, beta_ref,
                 en_ref, sig_ref, mf_ref, mb_ref):
    x = _bn_apply(t_ref[...].astype(jnp.float32), stats_ref, float(N_EDGES),
                  gamma_ref, beta_ref)
    en = e_ref[...] + jnp.maximum(x, 0.0)
    en_ref[...] = en
    sig = jax.nn.sigmoid(en)
    sig_ref[...] = sig
    mf_ref[...] = _unpack_lo(g3_ref[...]) * sig
    mb_ref[...] = _unpack_lo(g4_ref[...]) * sig


def _edge_b(t, e, gs, gd, stats, gamma, beta):
    return pl.pallas_call(
        _edge_b_body,
        grid=(EGRID,),
        in_specs=[
            pl.BlockSpec((EB, HIDDEN), lambda i: (i, 0)),
            pl.BlockSpec((EB, HIDDEN), lambda i: (i, 0)),
            pl.BlockSpec((EB, HIDDEN), lambda i: (i, 0)),
            pl.BlockSpec((EB, HIDDEN), lambda i: (i, 0)),
            pl.BlockSpec((8, HIDDEN), lambda i: (0, 0)),
            pl.BlockSpec((1, HIDDEN), lambda i: (0, 0)),
            pl.BlockSpec((1, HIDDEN), lambda i: (0, 0)),
        ],
        out_specs=[pl.BlockSpec((EB, HIDDEN), lambda i: (i, 0))] * 4,
        out_shape=[jax.ShapeDtypeStruct((N_EDGES, HIDDEN), jnp.float32)] * 4,
    )(t, e, gs, gd, stats, gamma, beta)


def _node_a_body(a1_ref, shf_ref, sf_ref, shb_ref, sb_ref, ht_ref, stats_ref):
    ht = (
        a1_ref[...]
        + shf_ref[...] / (sf_ref[...] + 1e-6)
        + shb_ref[...] / (sb_ref[...] + 1e-6)
    )
    ht_ref[...] = ht
    part = jnp.concatenate(
        [
            jnp.sum(ht, axis=0, keepdims=True),
            jnp.sum(ht * ht, axis=0, keepdims=True),
            jnp.zeros((6, HIDDEN), jnp.float32),
        ],
        axis=0,
    )

    @pl.when(pl.program_id(0) == 0)
    def _init():
        stats_ref[...] = part

    @pl.when(pl.program_id(0) != 0)
    def _acc():
        stats_ref[...] += part


def _node_a(a1h, shf, sf, shb, sb):
    return pl.pallas_call(
        _node_a_body,
        grid=(NGRID,),
        in_specs=[pl.BlockSpec((NB, HIDDEN), lambda i: (i, 0))] * 5,
        out_specs=[
            pl.BlockSpec((NB, HIDDEN), lambda i: (i, 0)),
            pl.BlockSpec((8, HIDDEN), lambda i: (0, 0)),
        ],
        out_shape=[
            jax.ShapeDtypeStruct((N_NODES, HIDDEN), jnp.float32),
            jax.ShapeDtypeStruct((8, HIDDEN), jnp.float32),
        ],
    )(a1h, shf, sf, shb, sb)


def _node_b_body(ht_ref, h_ref, stats_ref, gamma_ref, beta_ref, out_ref):
    x = _bn_apply(ht_ref[...], stats_ref, float(N_NODES), gamma_ref, beta_ref)
    out_ref[...] = h_ref[...] + jnp.maximum(x, 0.0)


def _node_b(ht, h, stats, gamma, beta):
    return pl.pallas_call(
        _node_b_body,
        grid=(NGRID,),
        in_specs=[
            pl.BlockSpec((NB, HIDDEN), lambda i: (i, 0)),
            pl.BlockSpec((NB, HIDDEN), lambda i: (i, 0)),
            pl.BlockSpec((8, HIDDEN), lambda i: (0, 0)),
            pl.BlockSpec((1, HIDDEN), lambda i: (0, 0)),
            pl.BlockSpec((1, HIDDEN), lambda i: (0, 0)),
        ],
        out_specs=pl.BlockSpec((NB, HIDDEN), lambda i: (i, 0)),
        out_shape=jax.ShapeDtypeStruct((N_NODES, HIDDEN), jnp.float32),
    )(ht, h, stats, gamma, beta)


_SC_MESH = plsc.VectorSubcoreMesh(core_axis_name="c", subcore_axis_name="s")


@functools.partial(
    pl.kernel,
    out_type=[
        jax.ShapeDtypeStruct((N_EDGES, HIDDEN), jnp.int32),
        jax.ShapeDtypeStruct((N_EDGES, HIDDEN), jnp.int32),
    ],
    mesh=_SC_MESH,
    scratch_types=[
        pltpu.VMEM((GC,), jnp.int32),
        pltpu.VMEM((GC,), jnp.int32),
        pltpu.VMEM((GC, HIDDEN), jnp.int32),
        pltpu.VMEM((GC, HIDDEN), jnp.int32),
        pltpu.SemaphoreType.DMA,
        pltpu.SemaphoreType.DMA,
    ],
)
def _sc_gather(tsrc_hbm, tdst_hbm, src_hbm, dst_hbm, gs_hbm, gd_hbm,
               idx1_v, idx2_v, rows1_v, rows2_v, sem1, sem2):
    wid = lax.axis_index("s") * NCORES + lax.axis_index("c")

    def chunk(k, _):
        cid = wid + k * NW

        @pl.when(cid < G_CHUNKS)
        def _do():
            eoff = pl.multiple_of(cid * GC, GC)
            pltpu.sync_copy(src_hbm.at[pl.ds(eoff, GC)], idx1_v)
            pltpu.sync_copy(dst_hbm.at[pl.ds(eoff, GC)], idx2_v)
            c1 = pltpu.async_copy(tsrc_hbm.at[idx1_v], rows1_v, sem1)
            c2 = pltpu.async_copy(tdst_hbm.at[idx2_v], rows2_v, sem2)
            c1.wait()
            pltpu.sync_copy(rows1_v, gs_hbm.at[pl.ds(eoff, GC)])
            c2.wait()
            pltpu.sync_copy(rows2_v, gd_hbm.at[pl.ds(eoff, GC)])

        return 0

    lax.fori_loop(0, G_ITERS, chunk, 0)


@functools.partial(
    pl.kernel,
    out_type=[jax.ShapeDtypeStruct((N_NODES, HIDDEN), jnp.float32)] * 4,
    mesh=_SC_MESH,
    scratch_types=[
        pltpu.VMEM((8, CPASS), jnp.float32),
        [pltpu.VMEM((SC_C,), jnp.int32)] * NSLOT,
        [pltpu.VMEM((SC_C, CPASS), jnp.float32)] * NSLOT,
        [pltpu.SemaphoreType.DMA] * NSLOT,
        [pltpu.SemaphoreType.DMA] * NSLOT,
        pltpu.VMEM_SHARED((N_NODES, CPASS), jnp.float32),
    ],
)
def _sc_scatter(msgf_hbm, msgb_hbm, sig_hbm, dst_hbm, src_hbm,
                shf_hbm, shb_hbm, sf_hbm, sb_hbm,
                zbuf_v, idx_s, buf_s, ldsem_s, adsem_s, acc_sh):
    cid = lax.axis_index("c")
    sid = lax.axis_index("s")
    r0 = pl.multiple_of(sid * RPT, 8)
    last = sid == NSUB - 1

    def zrow(i, _):
        for j in range(CPASS // 16):
            zbuf_v[i, pl.ds(j * 16, 16)] = jnp.zeros((16,), jnp.float32)
        return 0

    lax.fori_loop(0, 8, zrow, 0)
    nz = jnp.where(last, RPT_LAST // 8, RPT // 8)

    def _valid(k):
        return jnp.logical_and(k >= 0, sid + k * NSUB < S_CHUNKS)

    def _eoff(k):
        return pl.multiple_of((sid + k * NSUB) * SC_C, SC_C)

    for p in range(4):
        msg_hbm = msgf_hbm if p < 2 else msgb_hbm
        idx_hbm = dst_hbm if p < 2 else src_hbm
        outs = (shf_hbm, sf_hbm) if p < 2 else (shb_hbm, sb_hbm)
        c0 = (p % 2) * CPASS

        def zcopy(i, _):
            pltpu.sync_copy(zbuf_v, acc_sh.at[pl.ds(r0 + i * 8, 8)])
            return 0

        lax.fori_loop(0, nz, zcopy, 0)
        plsc.subcore_barrier()

        def _start_load(k, s, msg_hbm=msg_hbm, idx_hbm=idx_hbm, c0=c0):
            eoff = _eoff(k)
            pltpu.async_copy(idx_hbm.at[pl.ds(eoff, SC_C)], idx_s[s],
                             ldsem_s[s])

            @pl.when(cid == 0)
            def _ld_msg():
                pltpu.async_copy(
                    msg_hbm.at[pl.ds(eoff, SC_C), pl.ds(c0, CPASS)],
                    buf_s[s], ldsem_s[s])

            @pl.when(cid == 1)
            def _ld_sig():
                pltpu.async_copy(
                    sig_hbm.at[pl.ds(eoff, SC_C), pl.ds(c0, CPASS)],
                    buf_s[s], ldsem_s[s])

        def _wait_load(k, s, msg_hbm=msg_hbm, idx_hbm=idx_hbm, c0=c0):
            eoff = _eoff(k)
            pltpu.make_async_copy(idx_hbm.at[pl.ds(eoff, SC_C)], idx_s[s],
                                  ldsem_s[s]).wait()
            pltpu.make_async_copy(
                msg_hbm.at[pl.ds(eoff, SC_C), pl.ds(c0, CPASS)],
                buf_s[s], ldsem_s[s]).wait()

        def _wait_add(s):
            pltpu.make_async_copy(buf_s[s], acc_sh.at[idx_s[s]],
                                  adsem_s[s]).wait()

        def step(g, _, _start_load=_start_load, _wait_load=_wait_load):
            for s in range(NSLOT):
                k = g * NSLOT + s

                @pl.when(_valid(k - NSLOT))
                def _drain(s=s):
                    _wait_add(s)

                @pl.when(_valid(k))
                def _load(k=k, s=s):
                    _start_load(k, s)

                j = k - (NSLOT - 1)
                sj = (s + 1) % NSLOT

                @pl.when(_valid(j))
                def _process(j=j, sj=sj):
                    _wait_load(j, sj)
                    pltpu.async_copy(buf_s[sj], acc_sh.at[idx_s[sj]],
                                     adsem_s[sj], add=True)

            return 0

        n_groups = -(-(S_ITERS + NSLOT - 1) // NSLOT)
        lax.fori_loop(0, n_groups, step, 0)
        k_tail = NSLOT * (n_groups - 1)

        @pl.when(_valid(k_tail))
        def _drain_tail():
            _wait_add(k_tail % NSLOT)

        plsc.subcore_barrier()

        for c in range(NCORES):
            @pl.when(jnp.logical_and(cid == c, jnp.logical_not(last)))
            def _wb0(c=c, outs=outs, c0=c0):
                pltpu.sync_copy(
                    acc_sh.at[pl.ds(r0, RPT)],
                    outs[c].at[pl.ds(r0, RPT), pl.ds(c0, CPASS)])

            @pl.when(jnp.logical_and(cid == c, last))
            def _wb1(c=c, outs=outs, c0=c0):
                pltpu.sync_copy(
                    acc_sh.at[pl.ds(r0, RPT_LAST)],
                    outs[c].at[pl.ds(r0, RPT_LAST), pl.ds(c0, CPASS)])

        plsc.subcore_barrier()


def kernel(h, e, edge_index, W, b, bn_gamma, bn_beta):
    src, dst = edge_index[0], edge_index[1]

    for l in range(NUM_LAYERS):
        order = (0, 1, 3, 2, 4)
        wcat = jnp.concatenate([W[l, i].T for i in order], axis=1)
        bcat = jnp.concatenate([b[l, i] for i in order]).reshape(1, 5 * HIDDEN)
        a1h, tsrc, tdst = _node_proj(h, wcat, bcat)

        gs, gd = _sc_gather(tsrc, tdst, src, dst)

        t, estats = _edge_a(e, gs, gd, W[l, 5].T, b[l, 5].reshape(1, HIDDEN))
        e, sigma, msgf, msgb = _edge_b(
            t, e, gs, gd, estats,
            bn_gamma[l, 1].reshape(1, HIDDEN), bn_beta[l, 1].reshape(1, HIDDEN),
        )

        shf, shb, sf, sb = _sc_scatter(msgf, msgb, sigma, dst, src)

        ht, nstats = _node_a(a1h, shf, sf, shb, sb)
        h = _node_b(
            ht, h, nstats,
            bn_gamma[l, 0].reshape(1, HIDDEN), bn_beta[l, 0].reshape(1, HIDDEN),
        )

    return h, e
